# baseline (speedup 1.0000x reference)
import functools

import jax
import jax.numpy as jnp
from jax import lax
from jax.experimental import pallas as pl
from jax.experimental.pallas import tpu as pltpu
from jax.experimental.pallas import tpu_sc as plsc

NC = 2
NS = 16
NW = NC * NS

B = 16384
D = 64
HID = 32
ACT = 18

NPAIR = 500000
LANES = 16

ROWS_W = B // NW
CH = 128
NCH = ROWS_W // CH
NG = CH // LANES


@functools.cache
def _make_sc_gather():
    mesh = plsc.VectorSubcoreMesh(
        core_axis_name="c", subcore_axis_name="s", num_cores=NC, num_subcores=NS
    )

    @functools.partial(
        pl.kernel,
        out_type=jax.ShapeDtypeStruct((NW, NCH, CH, 128), jnp.float32),
        mesh=mesh,
        scratch_types=[
            pltpu.VMEM((NCH, CH), jnp.int32),
            pltpu.VMEM((NCH, CH), jnp.int32),
            pltpu.VMEM((NCH, CH, 128), jnp.float32),
            pltpu.VMEM((CH, 128), jnp.float32),
            pltpu.SemaphoreType.DMA,
        ],
        compiler_params=pltpu.CompilerParams(needs_layout_passes=False),
    )
    def _sc_gather(idx_hbm, table_hbm, out_hbm, idx_v, t_v, slabs, x_v, sem):
        wid = lax.axis_index("s") * NC + lax.axis_index("c")
        pltpu.sync_copy(idx_hbm.at[wid], idx_v)
        iota = lax.iota(jnp.int32, LANES)
        d16 = [jnp.full((LANES,), d, jnp.int32) for d in range(D)]

        for c in range(NCH):
            for g in range(NG):
                sl = pl.ds(g * LANES, LANES)
                t_v[c, sl] = idx_v[c, sl] >> 1

        descs = [
            pltpu.async_copy(table_hbm.at[t_v.at[c]], slabs.at[c], sem)
            for c in range(NCH)
        ]

        for c in range(NCH):
            descs[c].wait()
            slab = slabs.at[c]
            for g in range(NG):
                sl = pl.ds(g * LANES, LANES)
                base = g * LANES + iota
                q16 = (idx_v[c, sl] & 1) * D
                for d in range(D):
                    vals = plsc.load_gather(slab, [base, q16 + d16[d]])
                    plsc.store_scatter(x_v, [base, d16[d]], vals)
            pltpu.sync_copy(x_v, out_hbm.at[wid, c])

    return _sc_gather


def _mlp_body(x128_ref, w1_ref, b1_ref, w2_ref, b2_ref, eye_ref, yT_ref, xT_ref):
    x = x128_ref[:, :D]
    h = lax.dot_general(
        x, w1_ref[...], (((1,), (1,)), ((), ())),
        preferred_element_type=jnp.float32,
    )
    h = jnp.maximum(h + b1_ref[...], 0.0)
    yT = lax.dot_general(
        w2_ref[...], h, (((1,), (1,)), ((), ())),
        preferred_element_type=jnp.float32,
    )
    yT_ref[...] = yT + b2_ref[...]
    xT_ref[...] = lax.dot_general(
        eye_ref[...], x, (((1,), (1,)), ((), ())),
        preferred_element_type=jnp.float32,
    )


def _mlp(x128, W1, b1, W2, b2):
    BB = 2048
    grid = (B // BB,)
    return pl.pallas_call(
        _mlp_body,
        grid=grid,
        in_specs=[
            pl.BlockSpec((BB, 128), lambda i: (i, 0)),
            pl.BlockSpec((HID, D), lambda i: (0, 0)),
            pl.BlockSpec((1, HID), lambda i: (0, 0)),
            pl.BlockSpec((ACT, HID), lambda i: (0, 0)),
            pl.BlockSpec((ACT, 1), lambda i: (0, 0)),
            pl.BlockSpec((D, D), lambda i: (0, 0)),
        ],
        out_specs=[
            pl.BlockSpec((ACT, BB), lambda i: (0, i)),
            pl.BlockSpec((D, BB), lambda i: (0, i)),
        ],
        out_shape=[
            jax.ShapeDtypeStruct((ACT, B), jnp.float32),
            jax.ShapeDtypeStruct((D, B), jnp.float32),
        ],
    )(x128, W1, b1.reshape(1, HID), W2, b2.reshape(ACT, 1), jnp.eye(D, dtype=jnp.float32))


def kernel(states, table, W1, b1, W2, b2):
    idx = states.reshape(NW, NCH, CH)
    table2 = table.reshape(NPAIR, 128)
    x128 = _make_sc_gather()(idx, table2).reshape(B, 128)
    yT, xT = _mlp(x128, W1, b1, W2, b2)
    return (yT.T, xT.T)


# speedup vs baseline: 2.2183x; 2.2183x over previous
import functools

import jax
import jax.numpy as jnp
from jax import lax
from jax.experimental import pallas as pl
from jax.experimental.pallas import tpu as pltpu
from jax.experimental.pallas import tpu_sc as plsc

NC = 2
NS = 16
NW = NC * NS

B = 16384
D = 64
HID = 32
ACT = 18

SUB = 8
NT = 1000000 // SUB
LANES = 16

ROWS_W = B // NW
CH = 128
NCH = ROWS_W // CH
NG = CH // LANES


@functools.cache
def _make_sc_gather():
    mesh = plsc.VectorSubcoreMesh(
        core_axis_name="c", subcore_axis_name="s", num_cores=NC, num_subcores=NS
    )

    @functools.partial(
        pl.kernel,
        out_type=jax.ShapeDtypeStruct((NW, NCH, CH, 128), jnp.float32),
        mesh=mesh,
        scratch_types=[
            pltpu.VMEM((NCH, CH), jnp.int32),
            pltpu.VMEM((3, LANES, SUB, D), jnp.float32),
            pltpu.VMEM((CH, 128), jnp.float32),
            pltpu.SemaphoreType.DMA,
            pltpu.SemaphoreType.DMA,
            pltpu.SemaphoreType.DMA,
            pltpu.SemaphoreType.DMA,
        ],
        compiler_params=pltpu.CompilerParams(needs_layout_passes=False),
    )
    def _sc_gather(
        idx_hbm, table_hbm, out_hbm, idx_v, slabs, x_v, semA, semB, semC, semX
    ):
        wid = lax.axis_index("s") * NC + lax.axis_index("c")
        pltpu.sync_copy(idx_hbm.at[wid], idx_v)
        iota = lax.iota(jnp.int32, LANES)
        d16 = [jnp.full((LANES,), d, jnp.int32) for d in range(D)]
        sems = (semA, semB, semC)

        def fire(c, g):
            vec = idx_v[c, pl.ds(g * LANES, LANES)]
            tvec = vec >> 3
            svec = vec & 7
            descs = []
            for r in range(LANES):
                t = lax.reduce_max(jnp.where(iota == r, tvec, -1), axes=(0,))
                descs.append(
                    pltpu.async_copy(
                        table_hbm.at[t], slabs.at[g % 3, r], sems[g % 3]
                    )
                )
            return (g, svec, descs)

        def extract(g, svec, descs):
            for cp in descs:
                cp.wait()
            slab = slabs.at[g % 3]
            base = g * LANES + iota
            for d in range(D):
                vals = plsc.load_gather(slab, [iota, svec, d16[d]])
                plsc.store_scatter(x_v, [base, d16[d]], vals)

        def chunk(c, carry):
            @pl.when(c > 0)
            def _():
                pltpu.make_async_copy(
                    x_v, out_hbm.at[wid, c - 1], semX
                ).wait()

            infl = [fire(c, 0), fire(c, 1)]
            for g in range(NG):
                if g + 2 < NG:
                    infl.append(fire(c, g + 2))
                extract(*infl.pop(0))
            pltpu.async_copy(x_v, out_hbm.at[wid, c], semX)
            return carry

        lax.fori_loop(0, NCH, chunk, 0)
        pltpu.make_async_copy(x_v, out_hbm.at[wid, NCH - 1], semX).wait()

    return _sc_gather


def _mlp_body(x128_ref, w1_ref, b1_ref, w2_ref, b2_ref, eye_ref, yT_ref, xT_ref):
    x = x128_ref[:, :D]
    h = lax.dot_general(
        x, w1_ref[...], (((1,), (1,)), ((), ())),
        preferred_element_type=jnp.float32,
    )
    h = jnp.maximum(h + b1_ref[...], 0.0)
    yT = lax.dot_general(
        w2_ref[...], h, (((1,), (1,)), ((), ())),
        preferred_element_type=jnp.float32,
    )
    yT_ref[...] = yT + b2_ref[...]
    xT_ref[...] = lax.dot_general(
        eye_ref[...], x, (((1,), (1,)), ((), ())),
        preferred_element_type=jnp.float32,
    )


def _mlp(x128, W1, b1, W2, b2):
    BB = 2048
    grid = (B // BB,)
    return pl.pallas_call(
        _mlp_body,
        grid=grid,
        in_specs=[
            pl.BlockSpec((BB, 128), lambda i: (i, 0)),
            pl.BlockSpec((HID, D), lambda i: (0, 0)),
            pl.BlockSpec((1, HID), lambda i: (0, 0)),
            pl.BlockSpec((ACT, HID), lambda i: (0, 0)),
            pl.BlockSpec((ACT, 1), lambda i: (0, 0)),
            pl.BlockSpec((D, D), lambda i: (0, 0)),
        ],
        out_specs=[
            pl.BlockSpec((ACT, BB), lambda i: (0, i)),
            pl.BlockSpec((D, BB), lambda i: (0, i)),
        ],
        out_shape=[
            jax.ShapeDtypeStruct((ACT, B), jnp.float32),
            jax.ShapeDtypeStruct((D, B), jnp.float32),
        ],
    )(x128, W1, b1.reshape(1, HID), W2, b2.reshape(ACT, 1), jnp.eye(D, dtype=jnp.float32))


def kernel(states, table, W1, b1, W2, b2):
    idx = states.reshape(NW, NCH, CH)
    table3 = table.reshape(NT, SUB, D)
    x128 = _make_sc_gather()(idx, table3).reshape(B, 128)
    yT, xT = _mlp(x128, W1, b1, W2, b2)
    return (yT.T, xT.T)


# speedup vs baseline: 2.5769x; 1.1617x over previous
import functools

import jax
import jax.numpy as jnp
from jax import lax
from jax.experimental import pallas as pl
from jax.experimental.pallas import tpu as pltpu
from jax.experimental.pallas import tpu_sc as plsc

NC = 2
NS = 16
NW = NC * NS

B = 16384
D = 64
HID = 32
ACT = 18

LANES = 16

ROWS_W = B // NW
CH = 128
NCH = ROWS_W // CH
NG = CH // LANES


@functools.cache
def _make_sc_gather():
    mesh = plsc.VectorSubcoreMesh(
        core_axis_name="c", subcore_axis_name="s", num_cores=NC, num_subcores=NS
    )

    @functools.partial(
        pl.kernel,
        out_type=jax.ShapeDtypeStruct((NW, NCH, CH, D), jnp.float32),
        mesh=mesh,
        scratch_types=[
            pltpu.VMEM((NCH, CH), jnp.int32),
            pltpu.VMEM((2, CH, D), jnp.float32),
            pltpu.SemaphoreType.DMA,
            pltpu.SemaphoreType.DMA,
            pltpu.SemaphoreType.DMA,
        ],
        compiler_params=pltpu.CompilerParams(needs_layout_passes=False),
    )
    def _sc_gather(idx_hbm, table_hbm, out_hbm, idx_v, x_v, semA, semB, semX):
        wid = lax.axis_index("s") * NC + lax.axis_index("c")
        pltpu.sync_copy(idx_hbm.at[wid], idx_v)
        iota = lax.iota(jnp.int32, LANES)
        sems = (semA, semB)

        def fire(c):
            descs = []
            for g in range(NG):
                vec = idx_v[c, pl.ds(g * LANES, LANES)]
                for r in range(LANES):
                    i = lax.reduce_max(jnp.where(iota == r, vec, -1), axes=(0,))
                    descs.append(
                        pltpu.async_copy(
                            table_hbm.at[i >> 3, i & 7],
                            x_v.at[c % 2, g * LANES + r],
                            sems[c % 2],
                        )
                    )
            return descs

        pend = {}
        for c in range(NCH):
            if c >= 2:
                pltpu.make_async_copy(
                    x_v.at[c % 2], out_hbm.at[wid, c - 2], semX
                ).wait()
            pend[c] = fire(c)
            if c >= 1:
                for cp in pend.pop(c - 1):
                    cp.wait()
                pltpu.async_copy(
                    x_v.at[(c - 1) % 2], out_hbm.at[wid, c - 1], semX
                )
        for cp in pend.pop(NCH - 1):
            cp.wait()
        pltpu.async_copy(x_v.at[(NCH - 1) % 2], out_hbm.at[wid, NCH - 1], semX)
        for c in (NCH - 2, NCH - 1):
            pltpu.make_async_copy(
                x_v.at[c % 2], out_hbm.at[wid, c], semX
            ).wait()

    return _sc_gather


def _mlp_body(x_ref, w1_ref, b1_ref, w2_ref, b2_ref, eye_ref, yT_ref, xT_ref):
    x = x_ref[...]
    h = lax.dot_general(
        x, w1_ref[...], (((1,), (1,)), ((), ())),
        preferred_element_type=jnp.float32,
    )
    h = jnp.maximum(h + b1_ref[...], 0.0)
    yT = lax.dot_general(
        w2_ref[...], h, (((1,), (1,)), ((), ())),
        preferred_element_type=jnp.float32,
    )
    yT_ref[...] = yT + b2_ref[...]
    xT_ref[...] = lax.dot_general(
        eye_ref[...], x, (((1,), (1,)), ((), ())),
        preferred_element_type=jnp.float32,
    )


def _mlp(x, W1, b1, W2, b2):
    BB = 2048
    grid = (B // BB,)
    return pl.pallas_call(
        _mlp_body,
        grid=grid,
        in_specs=[
            pl.BlockSpec((BB, D), lambda i: (i, 0)),
            pl.BlockSpec((HID, D), lambda i: (0, 0)),
            pl.BlockSpec((1, HID), lambda i: (0, 0)),
            pl.BlockSpec((ACT, HID), lambda i: (0, 0)),
            pl.BlockSpec((ACT, 1), lambda i: (0, 0)),
            pl.BlockSpec((D, D), lambda i: (0, 0)),
        ],
        out_specs=[
            pl.BlockSpec((ACT, BB), lambda i: (0, i)),
            pl.BlockSpec((D, BB), lambda i: (0, i)),
        ],
        out_shape=[
            jax.ShapeDtypeStruct((ACT, B), jnp.float32),
            jax.ShapeDtypeStruct((D, B), jnp.float32),
        ],
    )(x, W1, b1.reshape(1, HID), W2, b2.reshape(ACT, 1), jnp.eye(D, dtype=jnp.float32))


def kernel(states, table, W1, b1, W2, b2):
    idx = states.reshape(NW, NCH, CH)
    table3 = table.reshape(1000000 // 8, 8, D)
    x = _make_sc_gather()(idx, table3).reshape(B, D)
    yT, xT = _mlp(x, W1, b1, W2, b2)
    return (yT.T, xT.T)


# speedup vs baseline: 2.5892x; 1.0048x over previous
import functools

import jax
import jax.numpy as jnp
from jax import lax
from jax.experimental import pallas as pl
from jax.experimental.pallas import tpu as pltpu
from jax.experimental.pallas import tpu_sc as plsc

NC = 2
NS = 16
NW = NC * NS

B = 16384
D = 64
HID = 32
ACT = 18

LANES = 16

ROWS_W = B // NW
CH = 256
NCH = ROWS_W // CH
NG = CH // LANES


@functools.cache
def _make_sc_gather():
    mesh = plsc.VectorSubcoreMesh(
        core_axis_name="c", subcore_axis_name="s", num_cores=NC, num_subcores=NS
    )

    @functools.partial(
        pl.kernel,
        out_type=jax.ShapeDtypeStruct((NW, NCH, CH, D), jnp.float32),
        mesh=mesh,
        scratch_types=[
            pltpu.VMEM((NCH, CH), jnp.int32),
            pltpu.VMEM((2, CH, D), jnp.float32),
            pltpu.SemaphoreType.DMA,
            pltpu.SemaphoreType.DMA,
            pltpu.SemaphoreType.DMA,
        ],
        compiler_params=pltpu.CompilerParams(needs_layout_passes=False),
    )
    def _sc_gather(idx_hbm, table_hbm, out_hbm, idx_v, x_v, semA, semB, semX):
        wid = lax.axis_index("s") * NC + lax.axis_index("c")
        pltpu.sync_copy(idx_hbm.at[wid], idx_v)
        iota = lax.iota(jnp.int32, LANES)
        sems = (semA, semB)

        def fire(c):
            descs = []
            for g in range(NG):
                vec = idx_v[c, pl.ds(g * LANES, LANES)]
                for r in range(LANES):
                    i = lax.reduce_max(jnp.where(iota == r, vec, -1), axes=(0,))
                    descs.append(
                        pltpu.async_copy(
                            table_hbm.at[i >> 3, i & 7],
                            x_v.at[c % 2, g * LANES + r],
                            sems[c % 2],
                        )
                    )
            return descs

        pend = {}
        for c in range(NCH):
            if c >= 2:
                pltpu.make_async_copy(
                    x_v.at[c % 2], out_hbm.at[wid, c - 2], semX
                ).wait()
            pend[c] = fire(c)
            if c >= 1:
                for cp in pend.pop(c - 1):
                    cp.wait()
                pltpu.async_copy(
                    x_v.at[(c - 1) % 2], out_hbm.at[wid, c - 1], semX
                )
        for cp in pend.pop(NCH - 1):
            cp.wait()
        pltpu.async_copy(x_v.at[(NCH - 1) % 2], out_hbm.at[wid, NCH - 1], semX)
        for c in (NCH - 2, NCH - 1):
            pltpu.make_async_copy(
                x_v.at[c % 2], out_hbm.at[wid, c], semX
            ).wait()

    return _sc_gather


def _mlp_body(x_ref, w1_ref, b1_ref, w2_ref, b2_ref, eye_ref, yT_ref, xT_ref):
    x = x_ref[...]
    h = lax.dot_general(
        x, w1_ref[...], (((1,), (1,)), ((), ())),
        preferred_element_type=jnp.float32,
    )
    h = jnp.maximum(h + b1_ref[...], 0.0)
    yT = lax.dot_general(
        w2_ref[...], h, (((1,), (1,)), ((), ())),
        preferred_element_type=jnp.float32,
    )
    yT_ref[...] = yT + b2_ref[...]
    xT_ref[...] = lax.dot_general(
        eye_ref[...], x, (((1,), (1,)), ((), ())),
        preferred_element_type=jnp.float32,
    )


def _mlp(x, W1, b1, W2, b2):
    BB = 4096
    grid = (B // BB,)
    return pl.pallas_call(
        _mlp_body,
        grid=grid,
        in_specs=[
            pl.BlockSpec((BB, D), lambda i: (i, 0)),
            pl.BlockSpec((HID, D), lambda i: (0, 0)),
            pl.BlockSpec((1, HID), lambda i: (0, 0)),
            pl.BlockSpec((ACT, HID), lambda i: (0, 0)),
            pl.BlockSpec((ACT, 1), lambda i: (0, 0)),
            pl.BlockSpec((D, D), lambda i: (0, 0)),
        ],
        out_specs=[
            pl.BlockSpec((ACT, BB), lambda i: (0, i)),
            pl.BlockSpec((D, BB), lambda i: (0, i)),
        ],
        out_shape=[
            jax.ShapeDtypeStruct((ACT, B), jnp.float32),
            jax.ShapeDtypeStruct((D, B), jnp.float32),
        ],
    )(x, W1, b1.reshape(1, HID), W2, b2.reshape(ACT, 1), jnp.eye(D, dtype=jnp.float32))


def kernel(states, table, W1, b1, W2, b2):
    idx = states.reshape(NW, NCH, CH)
    table3 = table.reshape(1000000 // 8, 8, D)
    x = _make_sc_gather()(idx, table3).reshape(B, D)
    yT, xT = _mlp(x, W1, b1, W2, b2)
    return (yT.T, xT.T)
